# baseline (speedup 1.0000x reference)
import functools

import jax
import jax.numpy as jnp
from jax import lax
from jax.experimental import pallas as pl
from jax.experimental.pallas import tpu as pltpu
from jax.experimental.pallas import tpu_sc as plsc

_N = 65536

_NC = 2
_NS = 16
_NW = _NC * _NS
_L = 16
_COLS = _N // _NW
_CHUNK = 3 * _COLS

_mesh = plsc.VectorSubcoreMesh(core_axis_name="c", subcore_axis_name="s")


@functools.partial(
    pl.kernel,
    mesh=_mesh,
    out_type=jax.ShapeDtypeStruct((3, _N), jnp.float32),
    scratch_types=[
        pltpu.VMEM((3, _COLS), jnp.float32),
        pltpu.VMEM((3, _COLS), jnp.int32),
    ],
)
def _add_sc(x_hbm, i_hbm, o_hbm, xv, iv):
    wid = lax.axis_index("s") * _NC + lax.axis_index("c")
    base = wid * _COLS
    pltpu.sync_copy(x_hbm.at[:, pl.ds(base, _COLS)], xv)
    pltpu.sync_copy(i_hbm.at[:, pl.ds(base, _COLS)], iv)

    def step(j, carry):
        s = pl.ds(j * _L, _L)
        for r in range(3):
            xv[r, s] = xv[r, s] + iv[r, s].astype(jnp.float32)
        return carry

    lax.fori_loop(0, _COLS // _L, step, 0)
    pltpu.sync_copy(xv, o_hbm.at[:, pl.ds(base, _COLS)])


def kernel(input_xyzs, query_xyz_index):
    out = _add_sc(input_xyzs.T, query_xyz_index.T)
    return out.T


# speedup vs baseline: 6.1088x; 6.1088x over previous
import jax
import jax.numpy as jnp
from jax.experimental import pallas as pl
from jax.experimental.pallas import tpu as pltpu

def _body(x_ref, i_ref, o_ref):
    o_ref[...] = x_ref[...] + i_ref[...].astype(jnp.float32)

def kernel(input_xyzs, query_xyz_index):
    x = input_xyzs.T
    i = query_xyz_index.T
    G = 4
    B = 65536 // G
    out = pl.pallas_call(
        _body,
        grid=(G,),
        in_specs=[
            pl.BlockSpec((3, B), lambda g: (0, g)),
            pl.BlockSpec((3, B), lambda g: (0, g)),
        ],
        out_specs=pl.BlockSpec((3, B), lambda g: (0, g)),
        out_shape=jax.ShapeDtypeStruct((3, 65536), jnp.float32),
        compiler_params=pltpu.CompilerParams(dimension_semantics=("arbitrary",)),
    )(x, i)
    return out.T


# speedup vs baseline: 9.0162x; 1.4759x over previous
import jax
import jax.numpy as jnp
from jax.experimental import pallas as pl
from jax.experimental.pallas import tpu as pltpu

def _body(x_ref, i_ref, o_ref):
    o_ref[...] = x_ref[...] + i_ref[...].astype(jnp.float32)

def kernel(input_xyzs, query_xyz_index):
    x = input_xyzs.T
    i = query_xyz_index.T
    G = 2
    B = 65536 // G
    out = pl.pallas_call(
        _body,
        grid=(G,),
        in_specs=[
            pl.BlockSpec((3, B), lambda g: (0, g)),
            pl.BlockSpec((3, B), lambda g: (0, g)),
        ],
        out_specs=pl.BlockSpec((3, B), lambda g: (0, g)),
        out_shape=jax.ShapeDtypeStruct((3, 65536), jnp.float32),
        compiler_params=pltpu.CompilerParams(dimension_semantics=("arbitrary",)),
    )(x, i)
    return out.T


# speedup vs baseline: 9.0797x; 1.0070x over previous
import jax
import jax.numpy as jnp
from jax.experimental import pallas as pl
from jax.experimental.pallas import tpu as pltpu

def _body(x_ref, i_ref, o_ref):
    o_ref[...] = x_ref[...] + i_ref[...].astype(jnp.float32)

def kernel(input_xyzs, query_xyz_index):
    x = input_xyzs.T
    i = query_xyz_index.T
    G = 2
    B = 65536 // G
    out = pl.pallas_call(
        _body,
        grid=(G,),
        in_specs=[
            pl.BlockSpec((3, B), lambda g: (0, g)),
            pl.BlockSpec((3, B), lambda g: (0, g)),
        ],
        out_specs=pl.BlockSpec((3, B), lambda g: (0, g)),
        out_shape=jax.ShapeDtypeStruct((3, 65536), jnp.float32),
        compiler_params=pltpu.CompilerParams(dimension_semantics=("parallel",)),
    )(x, i)
    return out.T
